# baseline (speedup 1.0000x reference)
import functools

import jax
import jax.numpy as jnp
from jax import lax
from jax.experimental import pallas as pl
from jax.experimental.pallas import tpu as pltpu
from jax.experimental.pallas import tpu_sc as plsc

NC = 2
NS = 16
NW = NC * NS

SB = 128
DEPTH = 4


def _sc_partial_sums(x, batch2d, n_rows, d, g):
    num_sb = n_rows // SB
    unif = num_sb // NW
    rem = num_sb - NW * unif
    assert rem == 4 and NW == 32, "assignment closed form assumes rem == 4"
    mesh = plsc.VectorSubcoreMesh(
        core_axis_name="c", subcore_axis_name="s", num_cores=NC, num_subcores=NS
    )
    rows_per_tile = g // NS
    zrows = 8

    @functools.partial(
        pl.kernel,
        out_type=jax.ShapeDtypeStruct((NC, g, d), jnp.float32),
        mesh=mesh,
        scratch_types=[
            pltpu.VMEM((unif + 10, 128), jnp.int32),
            pltpu.VMEM((DEPTH, SB, d), jnp.float32),
            pltpu.VMEM((zrows, d), jnp.float32),
            pltpu.VMEM_SHARED((g, d), jnp.float32),
        ]
        + [pltpu.SemaphoreType.DMA] * (2 * DEPTH),
    )
    def sc_kernel(x_hbm, b_hbm, out_hbm, ids_v, rows_v, zbuf_v, acc, *sems_all):
        cid = lax.axis_index("c")
        sid = lax.axis_index("s")
        wid = sid * NC + cid
        base = unif * wid + (wid >> 3)
        sems = sems_all[:DEPTH]
        ssems = sems_all[DEPTH:]

        zeros16 = jnp.zeros((16,), jnp.float32)

        @pl.loop(0, zrows)
        def _zero(i):
            for k in range(d // 16):
                zbuf_v[i, pl.ds(k * 16, 16)] = zeros16

        abase = pl.multiple_of(base & ~7, 8)
        off = base - abase
        pltpu.async_copy(b_hbm.at[pl.ds(abase, unif + 10)], ids_v, sems[0])
        for i in range(rows_per_tile // zrows):
            pltpu.sync_copy(
                zbuf_v, acc.at[pl.ds(sid * rows_per_tile + i * zrows, zrows)]
            )
        pltpu.make_async_copy(
            b_hbm.at[pl.ds(0, unif + 10)], ids_v, sems[0]
        ).wait()
        plsc.subcore_barrier()

        def start(i):
            b = i % DEPTH
            return pltpu.async_copy(
                x_hbm.at[pl.ds((base + i) * SB, SB)], rows_v.at[b], sems[b]
            )

        def scatter(i):
            b = i % DEPTH
            return pltpu.async_copy(
                rows_v.at[b], acc.at[ids_v.at[off + i]], ssems[b], add=True
            )

        descs = {0: start(0), 1: start(1)}
        sdescs = {}
        for i in range(unif):
            if i >= 2:
                sdescs.pop(i - 2).wait()
            if i + 2 < unif:
                descs[i + 2] = start(i + 2)
            descs.pop(i).wait()
            sdescs[i] = scatter(i)

        for i in sorted(sdescs):
            sdescs.pop(i).wait()

        @pl.when((wid & 7) == 7)
        def _tail():
            b = unif % DEPTH
            pltpu.sync_copy(
                x_hbm.at[pl.ds((base + unif) * SB, SB)], rows_v.at[b]
            )
            pltpu.sync_copy(rows_v.at[b], acc.at[ids_v.at[off + unif]], add=True)

        plsc.subcore_barrier()
        pltpu.sync_copy(
            acc.at[pl.ds(sid * rows_per_tile, rows_per_tile)],
            out_hbm.at[cid, pl.ds(sid * rows_per_tile, rows_per_tile)],
        )

    return sc_kernel(x, batch2d)


def _combine_body(p_ref, o_ref):
    o_ref[...] = p_ref[0] + p_ref[1]


def kernel(x, batch, ptr):
    n, d = x.shape
    g = int(ptr.shape[0]) - 1
    batch2d = batch.astype(jnp.int32).reshape(n // 128, 128)
    batch2d = jnp.pad(batch2d, ((0, 8), (0, 0)))
    partials = _sc_partial_sums(x, batch2d, n, d, g)
    out = pl.pallas_call(
        _combine_body,
        out_shape=jax.ShapeDtypeStruct((g, d), jnp.float32),
    )(partials)
    return out


# speedup vs baseline: 1.3975x; 1.3975x over previous
import functools

import jax
import jax.numpy as jnp
from jax import lax
from jax.experimental import pallas as pl
from jax.experimental.pallas import tpu as pltpu
from jax.experimental.pallas import tpu_sc as plsc

NC = 2
NS = 16
NW = NC * NS

SB = 128
DEPTH = 4


def _sc_partial_sums(x, batch2d, n_rows, d, g):
    num_sb = n_rows // SB
    unif = num_sb // NW
    tail_n = num_sb - NW * unif
    mesh = plsc.VectorSubcoreMesh(
        core_axis_name="c", subcore_axis_name="s", num_cores=NC, num_subcores=NS
    )
    rows_per_tile = g // NS
    zrows = 8

    @functools.partial(
        pl.kernel,
        out_type=jax.ShapeDtypeStruct((NC, g, d), jnp.float32),
        mesh=mesh,
        scratch_types=[
            pltpu.VMEM((DEPTH, 1, 128), jnp.int32),
            pltpu.VMEM((DEPTH, SB, d), jnp.float32),
            pltpu.VMEM((zrows, d), jnp.float32),
            pltpu.VMEM_SHARED((g, d), jnp.float32),
        ]
        + [pltpu.SemaphoreType.DMA] * (2 * DEPTH),
    )
    def sc_kernel(x_hbm, b_hbm, out_hbm, ids_v, rows_v, zbuf_v, acc, *sems_all):
        cid = lax.axis_index("c")
        sid = lax.axis_index("s")
        wid = sid * NC + cid
        sems = sems_all[:DEPTH]
        ssems = sems_all[DEPTH:]

        zeros16 = jnp.zeros((16,), jnp.float32)

        @pl.loop(0, zrows)
        def _zero(i):
            for k in range(d // 16):
                zbuf_v[i, pl.ds(k * 16, 16)] = zeros16

        for i in range(rows_per_tile // zrows):
            pltpu.sync_copy(
                zbuf_v, acc.at[pl.ds(sid * rows_per_tile + i * zrows, zrows)]
            )
        plsc.subcore_barrier()

        def start(i):
            sb = wid + NW * i
            b = i % DEPTH
            dr = pltpu.async_copy(
                x_hbm.at[pl.ds(sb * SB, SB)], rows_v.at[b], sems[b]
            )
            di = pltpu.async_copy(b_hbm.at[pl.ds(sb, 1)], ids_v.at[b], sems[b])
            return dr, di

        def scatter(i):
            b = i % DEPTH
            return pltpu.async_copy(
                rows_v.at[b], acc.at[ids_v.at[b, 0]], ssems[b], add=True
            )

        descs = {0: start(0), 1: start(1)}
        sdescs = {}
        for i in range(unif):
            if i >= 2:
                sdescs.pop(i - 2).wait()
            if i + 2 < unif:
                descs[i + 2] = start(i + 2)
            dr, di = descs.pop(i)
            dr.wait()
            di.wait()
            sdescs[i] = scatter(i)

        for i in sorted(sdescs):
            sdescs.pop(i).wait()

        @pl.when(wid < tail_n)
        def _tail():
            sb = wid + NW * unif
            b = unif % DEPTH
            pltpu.sync_copy(x_hbm.at[pl.ds(sb * SB, SB)], rows_v.at[b])
            pltpu.sync_copy(b_hbm.at[pl.ds(sb, 1)], ids_v.at[b])
            pltpu.sync_copy(rows_v.at[b], acc.at[ids_v.at[b, 0]], add=True)

        plsc.subcore_barrier()
        pltpu.sync_copy(
            acc.at[pl.ds(sid * rows_per_tile, rows_per_tile)],
            out_hbm.at[cid, pl.ds(sid * rows_per_tile, rows_per_tile)],
        )

    return sc_kernel(x, batch2d)


def _combine_body(p_ref, o_ref):
    o_ref[...] = p_ref[0] + p_ref[1]


def kernel(x, batch, ptr):
    n, d = x.shape
    g = int(ptr.shape[0]) - 1
    batch2d = batch.astype(jnp.int32).reshape(n // 128, 128)
    partials = _sc_partial_sums(x, batch2d, n, d, g)
    out = pl.pallas_call(
        _combine_body,
        out_shape=jax.ShapeDtypeStruct((g, d), jnp.float32),
    )(partials)
    return out
